# Initial kernel scaffold; baseline (speedup 1.0000x reference)
import jax
import jax.numpy as jnp
from jax.experimental import pallas as pl


def kernel(v_0, v_1, v_2):
    raise NotImplementedError("write your pallas kernel here")


# speedup vs baseline: 29.3787x; 29.3787x over previous
import functools

import jax
import jax.numpy as jnp
from jax import lax
from jax.experimental import pallas as pl
from jax.experimental.pallas import tpu as pltpu
from jax.experimental.pallas import tpu_sc as plsc

L = 16
NC = 2
NS = 16
NW = NC * NS

B_ROWS = 4096
B_N = 8192
B_RPT = B_ROWS // NW
B_BUF = 4
A_ROWS = 64
A_N = 32768
A_RPT = A_ROWS // NW
A_SUB = A_N // B_N
C_ROWS = 32768
C_COLS = 64
C_RPS = C_ROWS // NS
C_BUF = 512
C_K = 3

NEG_INF = float("-inf")
POS_INF = float("inf")
IMAX = 2**31 - 1


def _insert(tv, ti, v, vi, largest):
  k = len(tv)
  for i in range(k):
    enter = (v > tv[i]) if largest else (v < tv[i])
    nt = jnp.where(enter, v, tv[i])
    ni = jnp.where(enter, vi, ti[i])
    dv = jnp.where(enter, tv[i], v)
    di = jnp.where(enter, ti[i], vi)
    tv[i], ti[i] = nt, ni
    v, vi = dv, di
  return tv, ti


def _scan_chunks(load, nchunks, carry, idx0, largest, lane):
  k = len(carry[0])

  def body(c, cr):
    tv = list(cr[0])
    ti = list(cr[1])
    v = load(c)
    vi = (idx0 + c * L) + lane
    tv, ti = _insert(tv, ti, v, vi, largest)
    return (tuple(tv), tuple(ti))

  return lax.fori_loop(0, nchunks, body, carry)


def _fresh_carry(k, largest):
  init = NEG_INF if largest else POS_INF
  return (tuple(jnp.full((L,), init, jnp.float32) for _ in range(k)),
          tuple(jnp.zeros((L,), jnp.int32) for _ in range(k)))


def _all_max(x):
  return plsc.cummax(lax.rev(plsc.cummax(x), (0,)))


def _all_min(x):
  return -_all_max(-x)


def _merge_row(carry, k, largest, lane):
  vals = list(carry[0])
  idxs = list(carry[1])
  sentinel = NEG_INF if largest else POS_INF
  red = jnp.maximum if largest else jnp.minimum
  outv = jnp.zeros((L,), jnp.float32)
  outi = jnp.zeros((L,), jnp.int32)
  for j in range(k):
    m = vals[0]
    for i in range(1, k):
      m = red(m, vals[i])
    s = _all_max(m) if largest else _all_min(m)
    cand = jnp.full((L,), IMAX, jnp.int32)
    for i in range(k):
      cand = jnp.where(vals[i] == s, jnp.minimum(cand, idxs[i]), cand)
    gi = _all_min(cand)
    outv = jnp.where(lane == j, s, outv)
    outi = jnp.where(lane == j, gi, outi)
    for i in range(k):
      hit = (vals[i] == s) & (idxs[i] == gi)
      vals[i] = jnp.where(hit, jnp.float32(sentinel), vals[i])
  return outv, outi


def _sc_body(v0h, v1h, v2h,
             outAv, outAi, outBv, outBi, outCv, outCi,
             bbuf, cbuf, resBv, resBi, resAv, resAi,
             pbufv, pbufi, mrgv, mrgi, stgv, stgi,
             shv, shi):
  c = lax.axis_index("c")
  s = lax.axis_index("s")
  wid = c * NS + s
  lane = lax.iota(jnp.int32, L)

  baseB = wid * B_RPT

  def b_chunk(ch, _):
    r0 = baseB + ch * B_BUF
    pltpu.sync_copy(v1h.at[pl.ds(r0, B_BUF), :], bbuf)

    def b_row(r, _):
      carry = _scan_chunks(lambda cc: bbuf[r, pl.ds(cc * L, L)],
                           B_N // L, _fresh_carry(4, False), 0, False, lane)
      rr = ch * B_BUF + r
      outv, outi = _merge_row(carry, 4, False, lane)
      resBv[rr, :] = outv
      resBi[rr, :] = outi
      return 0

    lax.fori_loop(0, B_BUF, b_row, 0)
    return 0

  lax.fori_loop(0, B_RPT // B_BUF, b_chunk, 0)
  pltpu.sync_copy(resBv, outBv.at[pl.ds(baseB, B_RPT), :])
  pltpu.sync_copy(resBi, outBi.at[pl.ds(baseB, B_RPT), :])

  baseA = wid * A_RPT

  def a_row(r, _):
    row = baseA + r
    carry = _fresh_carry(2, True)
    for q in range(A_SUB):
      pltpu.sync_copy(v0h.at[row, pl.ds(q * B_N, B_N)], bbuf.at[q])
    for q in range(A_SUB):
      carry = _scan_chunks(lambda cc, q=q: bbuf[q, pl.ds(cc * L, L)],
                           B_N // L, carry, q * B_N, True, lane)

    outv, outi = _merge_row(carry, 2, True, lane)
    resAv[r, :] = outv
    resAi[r, :] = outi
    return 0

  lax.fori_loop(0, A_RPT, a_row, 0)
  pltpu.sync_copy(resAv, outAv.at[pl.ds(baseA, A_RPT), :])
  pltpu.sync_copy(resAi, outAi.at[pl.ds(baseA, A_RPT), :])

  rbase = s * C_RPS
  coff = pl.multiple_of(c * 32, 8)
  carries = [_fresh_carry(C_K, True) for _ in range(2)]

  def c_chunk(ch, cr):
    r0 = rbase + ch * C_BUF
    pltpu.sync_copy(v2h.at[pl.ds(r0, C_BUF), :], cbuf)

    def c_row(r, cr2):
      (tv0, ti0), (tv1, ti1) = cr2
      row = r0 + r
      rowvec = jnp.full((L,), 0, jnp.int32) + row
      v0 = cbuf[r, pl.ds(coff, L)]
      v1 = cbuf[r, pl.ds(coff + L, L)]
      tv0, ti0 = _insert(list(tv0), list(ti0), v0, rowvec, True)
      tv1, ti1 = _insert(list(tv1), list(ti1), v1, rowvec, True)
      return ((tuple(tv0), tuple(ti0)), (tuple(tv1), tuple(ti1)))

    return lax.fori_loop(0, C_BUF, c_row, cr)

  carries = lax.fori_loop(0, C_RPS // C_BUF, c_chunk, tuple(carries))

  for g in range(2):
    tv, ti = carries[g]
    for j in range(C_K):
      pbufv[g, j, :] = tv[j]
      pbufi[g, j, :] = ti[j]
  pltpu.sync_copy(pbufv, shv.at[s])
  pltpu.sync_copy(pbufi, shi.at[s])
  plsc.subcore_barrier()

  @pl.when(s < 2)
  def _():
    g = s
    pltpu.sync_copy(shv, mrgv)
    pltpu.sync_copy(shi, mrgi)

    def m_tile(t, cr):
      tv = list(cr[0])
      ti = list(cr[1])
      for j in range(C_K):
        v = mrgv[t, g, j, :]
        vi = mrgi[t, g, j, :]
        tv, ti = _insert(tv, ti, v, vi, True)
      return (tuple(tv), tuple(ti))

    tv, ti = lax.fori_loop(0, NS, m_tile, _fresh_carry(C_K, True))
    for j in range(C_K):
      stgv[j, :] = tv[j]
      stgi[j, :] = ti[j]
    gout = c * 2 + g
    pltpu.sync_copy(stgv, outCv.at[gout])
    pltpu.sync_copy(stgi, outCi.at[gout])


@jax.jit
def kernel(v_0, v_1, v_2):
  v1r = v_1.reshape(B_ROWS, B_N)
  mesh = plsc.VectorSubcoreMesh(core_axis_name="c", subcore_axis_name="s")
  f32, i32 = jnp.float32, jnp.int32
  outs = pl.kernel(
      _sc_body,
      out_type=[
          jax.ShapeDtypeStruct((A_ROWS, L), f32),
          jax.ShapeDtypeStruct((A_ROWS, L), i32),
          jax.ShapeDtypeStruct((B_ROWS, L), f32),
          jax.ShapeDtypeStruct((B_ROWS, L), i32),
          jax.ShapeDtypeStruct((4, C_K, L), f32),
          jax.ShapeDtypeStruct((4, C_K, L), i32),
      ],
      mesh=mesh,
      compiler_params=pltpu.CompilerParams(
          needs_layout_passes=False, use_tc_tiling_on_sc=False),
      scratch_types=[
          pltpu.VMEM((B_BUF, B_N), f32),
          pltpu.VMEM((C_BUF, C_COLS), f32),
          pltpu.VMEM((B_RPT, L), f32),
          pltpu.VMEM((B_RPT, L), i32),
          pltpu.VMEM((A_RPT, L), f32),
          pltpu.VMEM((A_RPT, L), i32),
          pltpu.VMEM((2, C_K, L), f32),
          pltpu.VMEM((2, C_K, L), i32),
          pltpu.VMEM((NS, 2, C_K, L), f32),
          pltpu.VMEM((NS, 2, C_K, L), i32),
          pltpu.VMEM((C_K, L), f32),
          pltpu.VMEM((C_K, L), i32),
          pltpu.VMEM_SHARED((NS, 2, C_K, L), f32),
          pltpu.VMEM_SHARED((NS, 2, C_K, L), i32),
      ],
  )(v_0, v1r, v_2)
  outAv, outAi, outBv, outBi, outCv, outCi = outs
  v4 = outAv[:, :2]
  v5 = outAi[:, :2]
  v7 = outBv[:, :4].reshape(8, 16, 32, 4)
  v8 = outBi[:, :4].reshape(8, 16, 32, 4)
  v10 = outCv.transpose(1, 0, 2).reshape(C_K, C_COLS)
  v11 = outCi.transpose(1, 0, 2).reshape(C_K, C_COLS)
  return (v4, v5, v7, v8, v10, v11)


# speedup vs baseline: 41.8047x; 1.4230x over previous
import functools

import jax
import jax.numpy as jnp
from jax import lax
from jax.experimental import pallas as pl
from jax.experimental.pallas import tpu as pltpu
from jax.experimental.pallas import tpu_sc as plsc

L = 16
NC = 2
NS = 16
NW = NC * NS

B_ROWS = 4096
B_N = 8192
B_RPT = B_ROWS // NW
B_BUF = 4
A_ROWS = 64
A_N = 32768
A_RPT = A_ROWS // NW
A_SUB = A_N // B_N
C_ROWS = 32768
C_COLS = 64
C_RPS = C_ROWS // NS
C_BUF = 512
C_K = 3

NEG_INF = float("-inf")
POS_INF = float("inf")
IMAX = 2**31 - 1


def _insert(tv, ti, v, vi, largest):
  k = len(tv)
  for i in range(k):
    enter = (v > tv[i]) if largest else (v < tv[i])
    nt = jnp.where(enter, v, tv[i])
    ni = jnp.where(enter, vi, ti[i])
    dv = jnp.where(enter, tv[i], v)
    di = jnp.where(enter, ti[i], vi)
    tv[i], ti[i] = nt, ni
    v, vi = dv, di
  return tv, ti


def _threshold(m, k, largest, lane):
  srt, _ = plsc.sort_key_val(m, m)
  pos = (L - k) if largest else (k - 1)
  if largest:
    return _all_max(jnp.where(lane == pos, srt, jnp.float32(NEG_INF)))
  return _all_min(jnp.where(lane == pos, srt, jnp.float32(POS_INF)))


GRP = 32


def _group_min_pass(load, nchunks, largest, lane):
  red = jnp.maximum if largest else jnp.minimum
  init = jnp.float32(NEG_INF if largest else POS_INF)

  def body(g, carry):
    m, gmv = carry
    gm = load(g * GRP)
    for i in range(1, GRP):
      gm = red(gm, load(g * GRP + i))
    gext = _all_max(gm) if largest else _all_min(gm)
    gmv = jnp.where(lane == g, gext, gmv)
    return (red(m, gm), gmv)

  return lax.fori_loop(0, nchunks // GRP, body,
                       (jnp.full((L,), init, jnp.float32),
                        jnp.full((L,), init, jnp.float32)))


def _guarded_pass(load, carry, idx0, largest, lane, thrv, gmv):
  done = jnp.float32(NEG_INF if largest else POS_INF)

  def mask_of(g):
    return (g >= thrv) if largest else (g <= thrv)

  def cond(cr):
    return jnp.any(mask_of(cr[0]))

  def body(cr):
    gmv2 = cr[0]
    tv = list(cr[1])
    ti = list(cr[2])
    gl = plsc.all_reduce_ffs(mask_of(gmv2))
    g_s = gl[0]
    for i in range(GRP):
      cc = g_s * GRP + i
      vi = (idx0 + cc * L) + lane
      tv, ti = _insert(tv, ti, load(cc), vi, largest)
    gmv2 = jnp.where(lane == gl, done, gmv2)
    return (gmv2, tuple(tv), tuple(ti))

  out = lax.while_loop(cond, body, (gmv, carry[0], carry[1]))
  return (out[1], out[2])


def _fresh_carry(k, largest):
  init = NEG_INF if largest else POS_INF
  return (tuple(jnp.full((L,), init, jnp.float32) for _ in range(k)),
          tuple(jnp.zeros((L,), jnp.int32) for _ in range(k)))


def _all_max(x):
  return plsc.cummax(lax.rev(plsc.cummax(x), (0,)))


def _all_min(x):
  return -_all_max(-x)


def _merge_row(carry, k, largest, lane):
  vals = list(carry[0])
  idxs = list(carry[1])
  sentinel = NEG_INF if largest else POS_INF
  red = jnp.maximum if largest else jnp.minimum
  outv = jnp.zeros((L,), jnp.float32)
  outi = jnp.zeros((L,), jnp.int32)
  for j in range(k):
    m = vals[0]
    for i in range(1, k):
      m = red(m, vals[i])
    s = _all_max(m) if largest else _all_min(m)
    cand = jnp.full((L,), IMAX, jnp.int32)
    for i in range(k):
      cand = jnp.where(vals[i] == s, jnp.minimum(cand, idxs[i]), cand)
    gi = _all_min(cand)
    outv = jnp.where(lane == j, s, outv)
    outi = jnp.where(lane == j, gi, outi)
    for i in range(k):
      hit = (vals[i] == s) & (idxs[i] == gi)
      vals[i] = jnp.where(hit, jnp.float32(sentinel), vals[i])
  return outv, outi


def _sc_body(v0h, v1h, v2h,
             outAv, outAi, outBv, outBi, outCv, outCi,
             bbuf, cbuf, resBv, resBi, resAv, resAi,
             pbufv, pbufi, mrgv, mrgi, stgv, stgi,
             shv, shi):
  c = lax.axis_index("c")
  s = lax.axis_index("s")
  wid = c * NS + s
  lane = lax.iota(jnp.int32, L)

  baseB = wid * B_RPT

  def b_chunk(ch, _):
    r0 = baseB + ch * B_BUF
    pltpu.sync_copy(v1h.at[pl.ds(r0, B_BUF), :], bbuf)

    def b_row(r, _):
      load = lambda cc: bbuf[r, pl.ds(cc * L, L)]
      m, gmv = _group_min_pass(load, B_N // L, False, lane)
      thrv = _threshold(m, 4, False, lane)
      carry = _guarded_pass(load, _fresh_carry(4, False),
                            0, False, lane, thrv, gmv)
      rr = ch * B_BUF + r
      outv, outi = _merge_row(carry, 4, False, lane)
      resBv[rr, :] = outv
      resBi[rr, :] = outi
      return 0

    lax.fori_loop(0, B_BUF, b_row, 0)
    return 0

  lax.fori_loop(0, B_RPT // B_BUF, b_chunk, 0)
  pltpu.sync_copy(resBv, outBv.at[pl.ds(baseB, B_RPT), :])
  pltpu.sync_copy(resBi, outBi.at[pl.ds(baseB, B_RPT), :])

  baseA = wid * A_RPT

  def a_row(r, _):
    row = baseA + r
    for q in range(A_SUB):
      pltpu.sync_copy(v0h.at[row, pl.ds(q * B_N, B_N)], bbuf.at[q])
    m = jnp.full((L,), jnp.float32(NEG_INF), jnp.float32)
    gmvs = []
    for q in range(A_SUB):
      mq, gmvq = _group_min_pass(
          lambda cc, q=q: bbuf[q, pl.ds(cc * L, L)], B_N // L, True, lane)
      m = jnp.maximum(m, mq)
      gmvs.append(gmvq)
    thrv = _threshold(m, 2, True, lane)
    carry = _fresh_carry(2, True)
    for q in range(A_SUB):
      carry = _guarded_pass(lambda cc, q=q: bbuf[q, pl.ds(cc * L, L)],
                            carry, q * B_N, True, lane, thrv, gmvs[q])

    outv, outi = _merge_row(carry, 2, True, lane)
    resAv[r, :] = outv
    resAi[r, :] = outi
    return 0

  lax.fori_loop(0, A_RPT, a_row, 0)
  pltpu.sync_copy(resAv, outAv.at[pl.ds(baseA, A_RPT), :])
  pltpu.sync_copy(resAi, outAi.at[pl.ds(baseA, A_RPT), :])

  rbase = s * C_RPS
  coff = pl.multiple_of(c * 32, 8)
  carries = [_fresh_carry(C_K, True) for _ in range(2)]

  def c_chunk(ch, cr):
    r0 = rbase + ch * C_BUF
    pltpu.sync_copy(v2h.at[pl.ds(r0, C_BUF), :], cbuf)

    def c_row(r, cr2):
      (tv0, ti0), (tv1, ti1) = cr2
      row = r0 + r
      rowvec = jnp.full((L,), 0, jnp.int32) + row
      v0 = cbuf[r, pl.ds(coff, L)]
      v1 = cbuf[r, pl.ds(coff + L, L)]
      tv0, ti0 = _insert(list(tv0), list(ti0), v0, rowvec, True)
      tv1, ti1 = _insert(list(tv1), list(ti1), v1, rowvec, True)
      return ((tuple(tv0), tuple(ti0)), (tuple(tv1), tuple(ti1)))

    return lax.fori_loop(0, C_BUF, c_row, cr)

  carries = lax.fori_loop(0, C_RPS // C_BUF, c_chunk, tuple(carries))

  for g in range(2):
    tv, ti = carries[g]
    for j in range(C_K):
      pbufv[g, j, :] = tv[j]
      pbufi[g, j, :] = ti[j]
  pltpu.sync_copy(pbufv, shv.at[s])
  pltpu.sync_copy(pbufi, shi.at[s])
  plsc.subcore_barrier()

  @pl.when(s < 2)
  def _():
    g = s
    pltpu.sync_copy(shv, mrgv)
    pltpu.sync_copy(shi, mrgi)

    def m_tile(t, cr):
      tv = list(cr[0])
      ti = list(cr[1])
      for j in range(C_K):
        v = mrgv[t, g, j, :]
        vi = mrgi[t, g, j, :]
        tv, ti = _insert(tv, ti, v, vi, True)
      return (tuple(tv), tuple(ti))

    tv, ti = lax.fori_loop(0, NS, m_tile, _fresh_carry(C_K, True))
    for j in range(C_K):
      stgv[j, :] = tv[j]
      stgi[j, :] = ti[j]
    gout = c * 2 + g
    pltpu.sync_copy(stgv, outCv.at[gout])
    pltpu.sync_copy(stgi, outCi.at[gout])


@jax.jit
def kernel(v_0, v_1, v_2):
  v1r = v_1.reshape(B_ROWS, B_N)
  mesh = plsc.VectorSubcoreMesh(core_axis_name="c", subcore_axis_name="s")
  f32, i32 = jnp.float32, jnp.int32
  outs = pl.kernel(
      _sc_body,
      out_type=[
          jax.ShapeDtypeStruct((A_ROWS, L), f32),
          jax.ShapeDtypeStruct((A_ROWS, L), i32),
          jax.ShapeDtypeStruct((B_ROWS, L), f32),
          jax.ShapeDtypeStruct((B_ROWS, L), i32),
          jax.ShapeDtypeStruct((4, C_K, L), f32),
          jax.ShapeDtypeStruct((4, C_K, L), i32),
      ],
      mesh=mesh,
      compiler_params=pltpu.CompilerParams(
          needs_layout_passes=False, use_tc_tiling_on_sc=False),
      scratch_types=[
          pltpu.VMEM((B_BUF, B_N), f32),
          pltpu.VMEM((C_BUF, C_COLS), f32),
          pltpu.VMEM((B_RPT, L), f32),
          pltpu.VMEM((B_RPT, L), i32),
          pltpu.VMEM((A_RPT, L), f32),
          pltpu.VMEM((A_RPT, L), i32),
          pltpu.VMEM((2, C_K, L), f32),
          pltpu.VMEM((2, C_K, L), i32),
          pltpu.VMEM((NS, 2, C_K, L), f32),
          pltpu.VMEM((NS, 2, C_K, L), i32),
          pltpu.VMEM((C_K, L), f32),
          pltpu.VMEM((C_K, L), i32),
          pltpu.VMEM_SHARED((NS, 2, C_K, L), f32),
          pltpu.VMEM_SHARED((NS, 2, C_K, L), i32),
      ],
  )(v_0, v1r, v_2)
  outAv, outAi, outBv, outBi, outCv, outCi = outs
  v4 = outAv[:, :2]
  v5 = outAi[:, :2]
  v7 = outBv[:, :4].reshape(8, 16, 32, 4)
  v8 = outBi[:, :4].reshape(8, 16, 32, 4)
  v10 = outCv.transpose(1, 0, 2).reshape(C_K, C_COLS)
  v11 = outCi.transpose(1, 0, 2).reshape(C_K, C_COLS)
  return (v4, v5, v7, v8, v10, v11)


# speedup vs baseline: 49.6381x; 1.1874x over previous
import functools

import jax
import jax.numpy as jnp
from jax import lax
from jax.experimental import pallas as pl
from jax.experimental.pallas import tpu as pltpu
from jax.experimental.pallas import tpu_sc as plsc

L = 16
NC = 2
NS = 16
NW = NC * NS

B_ROWS = 4096
B_N = 8192
B_RPT = B_ROWS // NW
B_BUF = 4
A_ROWS = 64
A_N = 32768
A_RPT = A_ROWS // NW
A_SUB = A_N // B_N
C_ROWS = 32768
C_COLS = 64
C_RPS = C_ROWS // NS
C_BUF = 256
C_K = 3

NEG_INF = float("-inf")
POS_INF = float("inf")
IMAX = 2**31 - 1


def _insert(tv, ti, v, vi, largest):
  k = len(tv)
  for i in range(k):
    enter = (v > tv[i]) if largest else (v < tv[i])
    nt = jnp.where(enter, v, tv[i])
    ni = jnp.where(enter, vi, ti[i])
    dv = jnp.where(enter, tv[i], v)
    di = jnp.where(enter, ti[i], vi)
    tv[i], ti[i] = nt, ni
    v, vi = dv, di
  return tv, ti


def _threshold(m, k, largest, lane):
  srt, _ = plsc.sort_key_val(m, m)
  pos = (L - k) if largest else (k - 1)
  if largest:
    return _all_max(jnp.where(lane == pos, srt, jnp.float32(NEG_INF)))
  return _all_min(jnp.where(lane == pos, srt, jnp.float32(POS_INF)))


GRP = 32


def _group_min_pass(load, nchunks, largest, lane):
  red = jnp.maximum if largest else jnp.minimum
  init = jnp.float32(NEG_INF if largest else POS_INF)

  def body(g, carry):
    m, gmv = carry
    gm = load(g * GRP)
    for i in range(1, GRP):
      gm = red(gm, load(g * GRP + i))
    gext = _all_max(gm) if largest else _all_min(gm)
    gmv = jnp.where(lane == g, gext, gmv)
    return (red(m, gm), gmv)

  return lax.fori_loop(0, nchunks // GRP, body,
                       (jnp.full((L,), init, jnp.float32),
                        jnp.full((L,), init, jnp.float32)))


def _guarded_pass(load, carry, idx0, largest, lane, thrv, gmv):
  done = jnp.float32(NEG_INF if largest else POS_INF)

  def mask_of(g):
    return (g >= thrv) if largest else (g <= thrv)

  def cond(cr):
    return jnp.any(mask_of(cr[0]))

  def body(cr):
    gmv2 = cr[0]
    tv = list(cr[1])
    ti = list(cr[2])
    gl = plsc.all_reduce_ffs(mask_of(gmv2))
    g_s = gl[0]
    for i in range(GRP):
      cc = g_s * GRP + i
      vi = (idx0 + cc * L) + lane
      tv, ti = _insert(tv, ti, load(cc), vi, largest)
    gmv2 = jnp.where(lane == gl, done, gmv2)
    return (gmv2, tuple(tv), tuple(ti))

  out = lax.while_loop(cond, body, (gmv, carry[0], carry[1]))
  return (out[1], out[2])


def _fresh_carry(k, largest):
  init = NEG_INF if largest else POS_INF
  return (tuple(jnp.full((L,), init, jnp.float32) for _ in range(k)),
          tuple(jnp.zeros((L,), jnp.int32) for _ in range(k)))


def _all_max(x):
  return plsc.cummax(lax.rev(plsc.cummax(x), (0,)))


def _all_min(x):
  return -_all_max(-x)


def _merge_row(carry, k, largest, lane):
  vals = list(carry[0])
  idxs = list(carry[1])
  sentinel = NEG_INF if largest else POS_INF
  red = jnp.maximum if largest else jnp.minimum
  outv = jnp.zeros((L,), jnp.float32)
  outi = jnp.zeros((L,), jnp.int32)
  for j in range(k):
    m = vals[0]
    for i in range(1, k):
      m = red(m, vals[i])
    s = _all_max(m) if largest else _all_min(m)
    cand = jnp.full((L,), IMAX, jnp.int32)
    for i in range(k):
      cand = jnp.where(vals[i] == s, jnp.minimum(cand, idxs[i]), cand)
    gi = _all_min(cand)
    outv = jnp.where(lane == j, s, outv)
    outi = jnp.where(lane == j, gi, outi)
    for i in range(k):
      hit = (vals[i] == s) & (idxs[i] == gi)
      vals[i] = jnp.where(hit, jnp.float32(sentinel), vals[i])
  return outv, outi


def _sc_body(v0h, v1h, v2h,
             outAv, outAi, outBv, outBi, outCv, outCi,
             bbuf, bbuf2, cbuf, cbuf2, resBv, resBi, resAv, resAi,
             pbufv, pbufi, mrgv, mrgi, stgv, stgi,
             shv, shi, sem0, sem1):
  c = lax.axis_index("c")
  s = lax.axis_index("s")
  wid = c * NS + s
  lane = lax.iota(jnp.int32, L)

  baseB = wid * B_RPT
  nchB = B_RPT // B_BUF

  def b_copy(ch, buf, sem):
    pltpu.async_copy(v1h.at[pl.ds(baseB + ch * B_BUF, B_BUF), :], buf, sem)

  def b_drain(buf, sem):
    pltpu.make_async_copy(v1h.at[pl.ds(baseB, B_BUF), :], buf, sem).wait()

  def b_process(buf, ch):
    def b_row(r, _):
      load = lambda cc: buf[r, pl.ds(cc * L, L)]
      m, gmv = _group_min_pass(load, B_N // L, False, lane)
      thrv = _threshold(m, 4, False, lane)
      carry = _guarded_pass(load, _fresh_carry(4, False),
                            0, False, lane, thrv, gmv)
      rr = ch * B_BUF + r
      outv, outi = _merge_row(carry, 4, False, lane)
      resBv[rr, :] = outv
      resBi[rr, :] = outi
      return 0

    lax.fori_loop(0, B_BUF, b_row, 0)

  b_copy(0, bbuf, sem0)
  b_copy(1, bbuf2, sem1)

  def b_pair(p, _):
    ch0 = 2 * p
    b_drain(bbuf, sem0)
    b_process(bbuf, ch0)

    @pl.when(ch0 + 2 < nchB)
    def _():
      b_copy(ch0 + 2, bbuf, sem0)

    b_drain(bbuf2, sem1)
    b_process(bbuf2, ch0 + 1)

    @pl.when(ch0 + 3 < nchB)
    def _():
      b_copy(ch0 + 3, bbuf2, sem1)

    return 0

  lax.fori_loop(0, nchB // 2, b_pair, 0)
  pltpu.sync_copy(resBv, outBv.at[pl.ds(baseB, B_RPT), :])
  pltpu.sync_copy(resBi, outBi.at[pl.ds(baseB, B_RPT), :])

  baseA = wid * A_RPT
  abufs = [bbuf, bbuf2]
  asems = [sem0, sem1]
  for r in range(A_RPT):
    for q in range(A_SUB):
      pltpu.async_copy(v0h.at[baseA + r, pl.ds(q * B_N, B_N)],
                       abufs[r].at[q], asems[r])
  for r in range(A_RPT):
    buf = abufs[r]
    pltpu.make_async_copy(v1h.at[pl.ds(0, B_BUF), :], buf, asems[r]).wait()
    m = jnp.full((L,), jnp.float32(NEG_INF), jnp.float32)
    gmvs = []
    for q in range(A_SUB):
      mq, gmvq = _group_min_pass(
          lambda cc, q=q: buf[q, pl.ds(cc * L, L)], B_N // L, True, lane)
      m = jnp.maximum(m, mq)
      gmvs.append(gmvq)
    thrv = _threshold(m, 2, True, lane)
    carry = _fresh_carry(2, True)
    for q in range(A_SUB):
      carry = _guarded_pass(lambda cc, q=q: buf[q, pl.ds(cc * L, L)],
                            carry, q * B_N, True, lane, thrv, gmvs[q])

    outv, outi = _merge_row(carry, 2, True, lane)
    resAv[r, :] = outv
    resAi[r, :] = outi

  pltpu.sync_copy(resAv, outAv.at[pl.ds(baseA, A_RPT), :])
  pltpu.sync_copy(resAi, outAi.at[pl.ds(baseA, A_RPT), :])

  rbase = s * C_RPS
  coff = pl.multiple_of(c * 32, 8)
  nchC = C_RPS // C_BUF

  def c_copy(ch, buf, sem):
    pltpu.async_copy(v2h.at[pl.ds(rbase + ch * C_BUF, C_BUF), :], buf, sem)

  def c_drain(buf, sem):
    pltpu.make_async_copy(v2h.at[pl.ds(0, C_BUF), :], buf, sem).wait()

  def c_process(buf, ch, cr):
    r0 = rbase + ch * C_BUF

    def c_row(r, cr2):
      (tv0, ti0), (tv1, ti1) = cr2
      row = r0 + r
      rowvec = jnp.full((L,), 0, jnp.int32) + row
      v0 = buf[r, pl.ds(coff, L)]
      v1 = buf[r, pl.ds(coff + L, L)]
      tv0, ti0 = _insert(list(tv0), list(ti0), v0, rowvec, True)
      tv1, ti1 = _insert(list(tv1), list(ti1), v1, rowvec, True)
      return ((tuple(tv0), tuple(ti0)), (tuple(tv1), tuple(ti1)))

    return lax.fori_loop(0, C_BUF, c_row, cr)

  c_copy(0, cbuf, sem0)
  c_copy(1, cbuf2, sem1)

  def c_pair(p, cr):
    ch0 = 2 * p
    c_drain(cbuf, sem0)
    cr = c_process(cbuf, ch0, cr)

    @pl.when(ch0 + 2 < nchC)
    def _():
      c_copy(ch0 + 2, cbuf, sem0)

    c_drain(cbuf2, sem1)
    cr = c_process(cbuf2, ch0 + 1, cr)

    @pl.when(ch0 + 3 < nchC)
    def _():
      c_copy(ch0 + 3, cbuf2, sem1)

    return cr

  carries = lax.fori_loop(
      0, nchC // 2, c_pair,
      tuple(_fresh_carry(C_K, True) for _ in range(2)))

  for g in range(2):
    tv, ti = carries[g]
    for j in range(C_K):
      pbufv[g, j, :] = tv[j]
      pbufi[g, j, :] = ti[j]
  pltpu.sync_copy(pbufv, shv.at[s])
  pltpu.sync_copy(pbufi, shi.at[s])
  plsc.subcore_barrier()

  @pl.when(s < 2)
  def _():
    g = s
    pltpu.sync_copy(shv, mrgv)
    pltpu.sync_copy(shi, mrgi)

    def m_tile(t, cr):
      tv = list(cr[0])
      ti = list(cr[1])
      for j in range(C_K):
        v = mrgv[t, g, j, :]
        vi = mrgi[t, g, j, :]
        tv, ti = _insert(tv, ti, v, vi, True)
      return (tuple(tv), tuple(ti))

    tv, ti = lax.fori_loop(0, NS, m_tile, _fresh_carry(C_K, True))
    for j in range(C_K):
      stgv[j, :] = tv[j]
      stgi[j, :] = ti[j]
    gout = c * 2 + g
    pltpu.sync_copy(stgv, outCv.at[gout])
    pltpu.sync_copy(stgi, outCi.at[gout])


@jax.jit
def kernel(v_0, v_1, v_2):
  v1r = v_1.reshape(B_ROWS, B_N)
  mesh = plsc.VectorSubcoreMesh(core_axis_name="c", subcore_axis_name="s")
  f32, i32 = jnp.float32, jnp.int32
  outs = pl.kernel(
      _sc_body,
      out_type=[
          jax.ShapeDtypeStruct((A_ROWS, L), f32),
          jax.ShapeDtypeStruct((A_ROWS, L), i32),
          jax.ShapeDtypeStruct((B_ROWS, L), f32),
          jax.ShapeDtypeStruct((B_ROWS, L), i32),
          jax.ShapeDtypeStruct((4, C_K, L), f32),
          jax.ShapeDtypeStruct((4, C_K, L), i32),
      ],
      mesh=mesh,
      compiler_params=pltpu.CompilerParams(
          needs_layout_passes=False, use_tc_tiling_on_sc=False),
      scratch_types=[
          pltpu.VMEM((B_BUF, B_N), f32),
          pltpu.VMEM((B_BUF, B_N), f32),
          pltpu.VMEM((C_BUF, C_COLS), f32),
          pltpu.VMEM((C_BUF, C_COLS), f32),
          pltpu.VMEM((B_RPT, L), f32),
          pltpu.VMEM((B_RPT, L), i32),
          pltpu.VMEM((A_RPT, L), f32),
          pltpu.VMEM((A_RPT, L), i32),
          pltpu.VMEM((2, C_K, L), f32),
          pltpu.VMEM((2, C_K, L), i32),
          pltpu.VMEM((NS, 2, C_K, L), f32),
          pltpu.VMEM((NS, 2, C_K, L), i32),
          pltpu.VMEM((C_K, L), f32),
          pltpu.VMEM((C_K, L), i32),
          pltpu.VMEM_SHARED((NS, 2, C_K, L), f32),
          pltpu.VMEM_SHARED((NS, 2, C_K, L), i32),
          pltpu.SemaphoreType.DMA,
          pltpu.SemaphoreType.DMA,
      ],
  )(v_0, v1r, v_2)
  outAv, outAi, outBv, outBi, outCv, outCi = outs
  v4 = outAv[:, :2]
  v5 = outAi[:, :2]
  v7 = outBv[:, :4].reshape(8, 16, 32, 4)
  v8 = outBi[:, :4].reshape(8, 16, 32, 4)
  v10 = outCv.transpose(1, 0, 2).reshape(C_K, C_COLS)
  v11 = outCi.transpose(1, 0, 2).reshape(C_K, C_COLS)
  return (v4, v5, v7, v8, v10, v11)


# speedup vs baseline: 53.4739x; 1.0773x over previous
import functools

import jax
import jax.numpy as jnp
from jax import lax
from jax.experimental import pallas as pl
from jax.experimental.pallas import tpu as pltpu
from jax.experimental.pallas import tpu_sc as plsc

L = 16
NC = 2
NS = 16
NW = NC * NS

B_ROWS = 4096
B_N = 8192
B_RPT = B_ROWS // NW
B_BUF = 4
A_ROWS = 64
A_N = 32768
A_RPT = A_ROWS // NW
A_SUB = A_N // B_N
C_ROWS = 32768
C_COLS = 64
C_RPS = C_ROWS // NS
C_BUF = 256
C_K = 3

NEG_INF = float("-inf")
POS_INF = float("inf")
IMAX = 2**31 - 1


def _insert(tv, ti, v, vi, largest):
  k = len(tv)
  for i in range(k):
    enter = (v > tv[i]) if largest else (v < tv[i])
    nt = jnp.where(enter, v, tv[i])
    ni = jnp.where(enter, vi, ti[i])
    dv = jnp.where(enter, tv[i], v)
    di = jnp.where(enter, ti[i], vi)
    tv[i], ti[i] = nt, ni
    v, vi = dv, di
  return tv, ti


def _threshold(m, k, largest, lane):
  srt, _ = plsc.sort_key_val(m, m)
  pos = (L - k) if largest else (k - 1)
  if largest:
    return _all_max(jnp.where(lane == pos, srt, jnp.float32(NEG_INF)))
  return _all_min(jnp.where(lane == pos, srt, jnp.float32(POS_INF)))


GRP = 16
SEG = 256


def _group_min_pass(load, nchunks, largest, lane):
  red = jnp.maximum if largest else jnp.minimum
  init = jnp.float32(NEG_INF if largest else POS_INF)

  def body(g, carry):
    m, gmv = carry
    gm = load(g * GRP)
    for i in range(1, GRP):
      gm = red(gm, load(g * GRP + i))
    gext = _all_max(gm) if largest else _all_min(gm)
    gmv = jnp.where(lane == g, gext, gmv)
    return (red(m, gm), gmv)

  return lax.fori_loop(0, nchunks // GRP, body,
                       (jnp.full((L,), init, jnp.float32),
                        jnp.full((L,), init, jnp.float32)))


def _guarded_pass(load, carry, idx0, largest, lane, thrv, gmv):
  done = jnp.float32(NEG_INF if largest else POS_INF)

  def mask_of(g):
    return (g >= thrv) if largest else (g <= thrv)

  def cond(cr):
    return jnp.any(mask_of(cr[0]))

  def body(cr):
    gmv2 = cr[0]
    tv = list(cr[1])
    ti = list(cr[2])
    gl = plsc.all_reduce_ffs(mask_of(gmv2))
    g_s = gl[0]
    for i in range(GRP):
      cc = g_s * GRP + i
      vi = (idx0 + cc * L) + lane
      tv, ti = _insert(tv, ti, load(cc), vi, largest)
    gmv2 = jnp.where(lane == gl, done, gmv2)
    return (gmv2, tuple(tv), tuple(ti))

  out = lax.while_loop(cond, body, (gmv, carry[0], carry[1]))
  return (out[1], out[2])


def _fresh_carry(k, largest):
  init = NEG_INF if largest else POS_INF
  return (tuple(jnp.full((L,), init, jnp.float32) for _ in range(k)),
          tuple(jnp.zeros((L,), jnp.int32) for _ in range(k)))


def _all_max(x):
  return plsc.cummax(lax.rev(plsc.cummax(x), (0,)))


def _all_min(x):
  return -_all_max(-x)


def _merge_row(carry, k, largest, lane):
  vals = list(carry[0])
  idxs = list(carry[1])
  sentinel = NEG_INF if largest else POS_INF
  red = jnp.maximum if largest else jnp.minimum
  outv = jnp.zeros((L,), jnp.float32)
  outi = jnp.zeros((L,), jnp.int32)
  for j in range(k):
    m = vals[0]
    for i in range(1, k):
      m = red(m, vals[i])
    s = _all_max(m) if largest else _all_min(m)
    cand = jnp.full((L,), IMAX, jnp.int32)
    for i in range(k):
      cand = jnp.where(vals[i] == s, jnp.minimum(cand, idxs[i]), cand)
    gi = _all_min(cand)
    outv = jnp.where(lane == j, s, outv)
    outi = jnp.where(lane == j, gi, outi)
    for i in range(k):
      hit = (vals[i] == s) & (idxs[i] == gi)
      vals[i] = jnp.where(hit, jnp.float32(sentinel), vals[i])
  return outv, outi


def _sc_body(v0h, v1h, v2h,
             outAv, outAi, outBv, outBi, outCv, outCi,
             bbuf, bbuf2, cbuf, cbuf2, resBv, resBi, resAv, resAi,
             pbufv, pbufi, mrgv, mrgi, stgv, stgi,
             shv, shi, sem0, sem1):
  c = lax.axis_index("c")
  s = lax.axis_index("s")
  wid = c * NS + s
  lane = lax.iota(jnp.int32, L)

  baseB = wid * B_RPT
  nchB = B_RPT // B_BUF

  def b_copy(ch, buf, sem):
    pltpu.async_copy(v1h.at[pl.ds(baseB + ch * B_BUF, B_BUF), :], buf, sem)

  def b_drain(buf, sem):
    pltpu.make_async_copy(v1h.at[pl.ds(baseB, B_BUF), :], buf, sem).wait()

  def b_process(buf, ch):
    def b_row(r, _):
      loads = [lambda cc, h=h: buf[r, pl.ds((h * SEG + cc) * L, L)]
               for h in range(B_N // L // SEG)]
      m = jnp.full((L,), jnp.float32(POS_INF), jnp.float32)
      gmvs = []
      for ld in loads:
        mh, gmvh = _group_min_pass(ld, SEG, False, lane)
        m = jnp.minimum(m, mh)
        gmvs.append(gmvh)
      thrv = _threshold(m, 4, False, lane)
      carry = _fresh_carry(4, False)
      for h, ld in enumerate(loads):
        carry = _guarded_pass(ld, carry, h * SEG * L, False, lane,
                              thrv, gmvs[h])
      rr = ch * B_BUF + r
      outv, outi = _merge_row(carry, 4, False, lane)
      resBv[rr, :] = outv
      resBi[rr, :] = outi
      return 0

    lax.fori_loop(0, B_BUF, b_row, 0)

  b_copy(0, bbuf, sem0)
  b_copy(1, bbuf2, sem1)

  def b_pair(p, _):
    ch0 = 2 * p
    b_drain(bbuf, sem0)
    b_process(bbuf, ch0)

    @pl.when(ch0 + 2 < nchB)
    def _():
      b_copy(ch0 + 2, bbuf, sem0)

    b_drain(bbuf2, sem1)
    b_process(bbuf2, ch0 + 1)

    @pl.when(ch0 + 3 < nchB)
    def _():
      b_copy(ch0 + 3, bbuf2, sem1)

    return 0

  lax.fori_loop(0, nchB // 2, b_pair, 0)
  pltpu.sync_copy(resBv, outBv.at[pl.ds(baseB, B_RPT), :])
  pltpu.sync_copy(resBi, outBi.at[pl.ds(baseB, B_RPT), :])

  baseA = wid * A_RPT
  abufs = [bbuf, bbuf2]
  asems = [sem0, sem1]
  for r in range(A_RPT):
    for q in range(A_SUB):
      pltpu.async_copy(v0h.at[baseA + r, pl.ds(q * B_N, B_N)],
                       abufs[r].at[q], asems[r])
  for r in range(A_RPT):
    buf = abufs[r]
    pltpu.make_async_copy(v1h.at[pl.ds(0, B_BUF), :], buf, asems[r]).wait()
    m = jnp.full((L,), jnp.float32(NEG_INF), jnp.float32)
    nseg = B_N // L // SEG
    loads = [lambda cc, q=q, h=h: buf[q, pl.ds((h * SEG + cc) * L, L)]
             for q in range(A_SUB) for h in range(nseg)]
    gmvs = []
    for ld in loads:
      mq, gmvq = _group_min_pass(ld, SEG, True, lane)
      m = jnp.maximum(m, mq)
      gmvs.append(gmvq)
    thrv = _threshold(m, 2, True, lane)
    carry = _fresh_carry(2, True)
    for i, ld in enumerate(loads):
      carry = _guarded_pass(ld, carry, i * SEG * L, True, lane, thrv,
                            gmvs[i])

    outv, outi = _merge_row(carry, 2, True, lane)
    resAv[r, :] = outv
    resAi[r, :] = outi

  pltpu.sync_copy(resAv, outAv.at[pl.ds(baseA, A_RPT), :])
  pltpu.sync_copy(resAi, outAi.at[pl.ds(baseA, A_RPT), :])

  rbase = s * C_RPS
  coff = pl.multiple_of(c * 32, 8)
  nchC = C_RPS // C_BUF

  def c_copy(ch, buf, sem):
    pltpu.async_copy(v2h.at[pl.ds(rbase + ch * C_BUF, C_BUF), :], buf, sem)

  def c_drain(buf, sem):
    pltpu.make_async_copy(v2h.at[pl.ds(0, C_BUF), :], buf, sem).wait()

  def c_process(buf, ch, cr):
    r0 = rbase + ch * C_BUF

    def c_row(r, cr2):
      (tv0, ti0), (tv1, ti1) = cr2
      row = r0 + r
      rowvec = jnp.full((L,), 0, jnp.int32) + row
      v0 = buf[r, pl.ds(coff, L)]
      v1 = buf[r, pl.ds(coff + L, L)]
      tv0, ti0 = _insert(list(tv0), list(ti0), v0, rowvec, True)
      tv1, ti1 = _insert(list(tv1), list(ti1), v1, rowvec, True)
      return ((tuple(tv0), tuple(ti0)), (tuple(tv1), tuple(ti1)))

    return lax.fori_loop(0, C_BUF, c_row, cr)

  c_copy(0, cbuf, sem0)
  c_copy(1, cbuf2, sem1)

  def c_pair(p, cr):
    ch0 = 2 * p
    c_drain(cbuf, sem0)
    cr = c_process(cbuf, ch0, cr)

    @pl.when(ch0 + 2 < nchC)
    def _():
      c_copy(ch0 + 2, cbuf, sem0)

    c_drain(cbuf2, sem1)
    cr = c_process(cbuf2, ch0 + 1, cr)

    @pl.when(ch0 + 3 < nchC)
    def _():
      c_copy(ch0 + 3, cbuf2, sem1)

    return cr

  carries = lax.fori_loop(
      0, nchC // 2, c_pair,
      tuple(_fresh_carry(C_K, True) for _ in range(2)))

  for g in range(2):
    tv, ti = carries[g]
    for j in range(C_K):
      pbufv[g, j, :] = tv[j]
      pbufi[g, j, :] = ti[j]
  pltpu.sync_copy(pbufv, shv.at[s])
  pltpu.sync_copy(pbufi, shi.at[s])
  plsc.subcore_barrier()

  @pl.when(s < 2)
  def _():
    g = s
    pltpu.sync_copy(shv, mrgv)
    pltpu.sync_copy(shi, mrgi)

    def m_tile(t, cr):
      tv = list(cr[0])
      ti = list(cr[1])
      for j in range(C_K):
        v = mrgv[t, g, j, :]
        vi = mrgi[t, g, j, :]
        tv, ti = _insert(tv, ti, v, vi, True)
      return (tuple(tv), tuple(ti))

    tv, ti = lax.fori_loop(0, NS, m_tile, _fresh_carry(C_K, True))
    for j in range(C_K):
      stgv[j, :] = tv[j]
      stgi[j, :] = ti[j]
    gout = c * 2 + g
    pltpu.sync_copy(stgv, outCv.at[gout])
    pltpu.sync_copy(stgi, outCi.at[gout])


@jax.jit
def kernel(v_0, v_1, v_2):
  v1r = v_1.reshape(B_ROWS, B_N)
  mesh = plsc.VectorSubcoreMesh(core_axis_name="c", subcore_axis_name="s")
  f32, i32 = jnp.float32, jnp.int32
  outs = pl.kernel(
      _sc_body,
      out_type=[
          jax.ShapeDtypeStruct((A_ROWS, L), f32),
          jax.ShapeDtypeStruct((A_ROWS, L), i32),
          jax.ShapeDtypeStruct((B_ROWS, L), f32),
          jax.ShapeDtypeStruct((B_ROWS, L), i32),
          jax.ShapeDtypeStruct((4, C_K, L), f32),
          jax.ShapeDtypeStruct((4, C_K, L), i32),
      ],
      mesh=mesh,
      compiler_params=pltpu.CompilerParams(
          needs_layout_passes=False, use_tc_tiling_on_sc=False),
      scratch_types=[
          pltpu.VMEM((B_BUF, B_N), f32),
          pltpu.VMEM((B_BUF, B_N), f32),
          pltpu.VMEM((C_BUF, C_COLS), f32),
          pltpu.VMEM((C_BUF, C_COLS), f32),
          pltpu.VMEM((B_RPT, L), f32),
          pltpu.VMEM((B_RPT, L), i32),
          pltpu.VMEM((A_RPT, L), f32),
          pltpu.VMEM((A_RPT, L), i32),
          pltpu.VMEM((2, C_K, L), f32),
          pltpu.VMEM((2, C_K, L), i32),
          pltpu.VMEM((NS, 2, C_K, L), f32),
          pltpu.VMEM((NS, 2, C_K, L), i32),
          pltpu.VMEM((C_K, L), f32),
          pltpu.VMEM((C_K, L), i32),
          pltpu.VMEM_SHARED((NS, 2, C_K, L), f32),
          pltpu.VMEM_SHARED((NS, 2, C_K, L), i32),
          pltpu.SemaphoreType.DMA,
          pltpu.SemaphoreType.DMA,
      ],
  )(v_0, v1r, v_2)
  outAv, outAi, outBv, outBi, outCv, outCi = outs
  v4 = outAv[:, :2]
  v5 = outAi[:, :2]
  v7 = outBv[:, :4].reshape(8, 16, 32, 4)
  v8 = outBi[:, :4].reshape(8, 16, 32, 4)
  v10 = outCv.transpose(1, 0, 2).reshape(C_K, C_COLS)
  v11 = outCi.transpose(1, 0, 2).reshape(C_K, C_COLS)
  return (v4, v5, v7, v8, v10, v11)


# speedup vs baseline: 53.6765x; 1.0038x over previous
import functools

import jax
import jax.numpy as jnp
from jax import lax
from jax.experimental import pallas as pl
from jax.experimental.pallas import tpu as pltpu
from jax.experimental.pallas import tpu_sc as plsc

L = 16
NC = 2
NS = 16
NW = NC * NS

B_ROWS = 4096
B_N = 8192
B_RPT = B_ROWS // NW
B_BUF = 4
A_ROWS = 64
A_N = 32768
A_RPT = A_ROWS // NW
A_SUB = A_N // B_N
C_ROWS = 32768
C_COLS = 64
C_RPS = C_ROWS // NS
C_BUF = 256
C_K = 3

NEG_INF = float("-inf")
POS_INF = float("inf")
IMAX = 2**31 - 1


def _insert(tv, ti, v, vi, largest):
  k = len(tv)
  for i in range(k):
    enter = (v > tv[i]) if largest else (v < tv[i])
    nt = jnp.where(enter, v, tv[i])
    ni = jnp.where(enter, vi, ti[i])
    dv = jnp.where(enter, tv[i], v)
    di = jnp.where(enter, ti[i], vi)
    tv[i], ti[i] = nt, ni
    v, vi = dv, di
  return tv, ti


def _threshold(m, k, largest, lane):
  srt, _ = plsc.sort_key_val(m, m)
  pos = (L - k) if largest else (k - 1)
  if largest:
    return _all_max(jnp.where(lane == pos, srt, jnp.float32(NEG_INF)))
  return _all_min(jnp.where(lane == pos, srt, jnp.float32(POS_INF)))


GRP = 16
SEG = 256


def _group_min_pass(load, nchunks, largest, lane):
  red = jnp.maximum if largest else jnp.minimum
  init = jnp.float32(NEG_INF if largest else POS_INF)

  def body(g, carry):
    m, gmv = carry
    gm = load(g * GRP)
    for i in range(1, GRP):
      gm = red(gm, load(g * GRP + i))
    gext = _all_max(gm) if largest else _all_min(gm)
    gmv = jnp.where(lane == g, gext, gmv)
    return (red(m, gm), gmv)

  return lax.fori_loop(0, nchunks // GRP, body,
                       (jnp.full((L,), init, jnp.float32),
                        jnp.full((L,), init, jnp.float32)))


def _guarded_pass(load, carry, idx0, largest, lane, thrv, gmv):
  done = jnp.float32(NEG_INF if largest else POS_INF)

  def mask_of(g):
    return (g >= thrv) if largest else (g <= thrv)

  def cond(cr):
    return plsc.all_reduce_population_count(mask_of(cr[0]))[0] > 0

  def body(cr):
    gmv2 = cr[0]
    tv = list(cr[1])
    ti = list(cr[2])
    gl = plsc.all_reduce_ffs(mask_of(gmv2))
    g_s = gl[0]
    for i in range(GRP):
      cc = g_s * GRP + i
      vi = (idx0 + cc * L) + lane
      tv, ti = _insert(tv, ti, load(cc), vi, largest)
    gmv2 = jnp.where(lane == gl, done, gmv2)
    return (gmv2, tuple(tv), tuple(ti))

  out = lax.while_loop(cond, body, (gmv, carry[0], carry[1]))
  return (out[1], out[2])


def _fresh_carry(k, largest):
  init = NEG_INF if largest else POS_INF
  return (tuple(jnp.full((L,), init, jnp.float32) for _ in range(k)),
          tuple(jnp.zeros((L,), jnp.int32) for _ in range(k)))


def _all_max(x):
  return plsc.cummax(lax.rev(plsc.cummax(x), (0,)))


def _all_min(x):
  return -_all_max(-x)


def _merge_row(carry, k, largest, lane):
  vals = list(carry[0])
  idxs = list(carry[1])
  sentinel = NEG_INF if largest else POS_INF
  red = jnp.maximum if largest else jnp.minimum
  outv = jnp.zeros((L,), jnp.float32)
  outi = jnp.zeros((L,), jnp.int32)
  for j in range(k):
    m = vals[0]
    if j > 0:
      for i in range(1, k):
        m = red(m, vals[i])
    s = _all_max(m) if largest else _all_min(m)
    cand = jnp.full((L,), IMAX, jnp.int32)
    for i in range(k):
      cand = jnp.where(vals[i] == s, jnp.minimum(cand, idxs[i]), cand)
    gi = _all_min(cand)
    outv = jnp.where(lane == j, s, outv)
    outi = jnp.where(lane == j, gi, outi)
    if j < k - 1:
      for i in range(k):
        hit = (vals[i] == s) & (idxs[i] == gi)
        vals[i] = jnp.where(hit, jnp.float32(sentinel), vals[i])
  return outv, outi


def _sc_body(v0h, v1h, v2h,
             outAv, outAi, outBv, outBi, outCv, outCi,
             bbuf, bbuf2, cbuf, cbuf2, resBv, resBi, resAv, resAi,
             pbufv, pbufi, mrgv, mrgi, stgv, stgi,
             shv, shi, sem0, sem1):
  c = lax.axis_index("c")
  s = lax.axis_index("s")
  wid = c * NS + s
  lane = lax.iota(jnp.int32, L)

  baseB = wid * B_RPT
  nchB = B_RPT // B_BUF

  def b_copy(ch, buf, sem):
    pltpu.async_copy(v1h.at[pl.ds(baseB + ch * B_BUF, B_BUF), :], buf, sem)

  def b_drain(buf, sem):
    pltpu.make_async_copy(v1h.at[pl.ds(baseB, B_BUF), :], buf, sem).wait()

  def b_process(buf, ch):
    def b_row(r, _):
      loads = [lambda cc, h=h: buf[r, pl.ds((h * SEG + cc) * L, L)]
               for h in range(B_N // L // SEG)]
      m = jnp.full((L,), jnp.float32(POS_INF), jnp.float32)
      gmvs = []
      for ld in loads:
        mh, gmvh = _group_min_pass(ld, SEG, False, lane)
        m = jnp.minimum(m, mh)
        gmvs.append(gmvh)
      thrv = _threshold(m, 4, False, lane)
      carry = _fresh_carry(4, False)
      for h, ld in enumerate(loads):
        carry = _guarded_pass(ld, carry, h * SEG * L, False, lane,
                              thrv, gmvs[h])
      rr = ch * B_BUF + r
      outv, outi = _merge_row(carry, 4, False, lane)
      resBv[rr, :] = outv
      resBi[rr, :] = outi
      return 0

    lax.fori_loop(0, B_BUF, b_row, 0)

  b_copy(0, bbuf, sem0)
  b_copy(1, bbuf2, sem1)

  def b_pair(p, _):
    ch0 = 2 * p
    b_drain(bbuf, sem0)
    b_process(bbuf, ch0)

    @pl.when(ch0 + 2 < nchB)
    def _():
      b_copy(ch0 + 2, bbuf, sem0)

    b_drain(bbuf2, sem1)
    b_process(bbuf2, ch0 + 1)

    @pl.when(ch0 + 3 < nchB)
    def _():
      b_copy(ch0 + 3, bbuf2, sem1)

    return 0

  lax.fori_loop(0, nchB // 2, b_pair, 0)
  pltpu.sync_copy(resBv, outBv.at[pl.ds(baseB, B_RPT), :])
  pltpu.sync_copy(resBi, outBi.at[pl.ds(baseB, B_RPT), :])

  baseA = wid * A_RPT
  abufs = [bbuf, bbuf2]
  asems = [sem0, sem1]
  for r in range(A_RPT):
    for q in range(A_SUB):
      pltpu.async_copy(v0h.at[baseA + r, pl.ds(q * B_N, B_N)],
                       abufs[r].at[q], asems[r])
  for r in range(A_RPT):
    buf = abufs[r]
    pltpu.make_async_copy(v1h.at[pl.ds(0, B_BUF), :], buf, asems[r]).wait()
    m = jnp.full((L,), jnp.float32(NEG_INF), jnp.float32)
    nseg = B_N // L // SEG
    loads = [lambda cc, q=q, h=h: buf[q, pl.ds((h * SEG + cc) * L, L)]
             for q in range(A_SUB) for h in range(nseg)]
    gmvs = []
    for ld in loads:
      mq, gmvq = _group_min_pass(ld, SEG, True, lane)
      m = jnp.maximum(m, mq)
      gmvs.append(gmvq)
    thrv = _threshold(m, 2, True, lane)
    carry = _fresh_carry(2, True)
    for i, ld in enumerate(loads):
      carry = _guarded_pass(ld, carry, i * SEG * L, True, lane, thrv,
                            gmvs[i])

    outv, outi = _merge_row(carry, 2, True, lane)
    resAv[r, :] = outv
    resAi[r, :] = outi

  pltpu.sync_copy(resAv, outAv.at[pl.ds(baseA, A_RPT), :])
  pltpu.sync_copy(resAi, outAi.at[pl.ds(baseA, A_RPT), :])

  rbase = s * C_RPS
  coff = pl.multiple_of(c * 32, 8)
  nchC = C_RPS // C_BUF

  def c_copy(ch, buf, sem):
    pltpu.async_copy(v2h.at[pl.ds(rbase + ch * C_BUF, C_BUF), :], buf, sem)

  def c_drain(buf, sem):
    pltpu.make_async_copy(v2h.at[pl.ds(0, C_BUF), :], buf, sem).wait()

  def c_process(buf, ch, cr):
    r0 = rbase + ch * C_BUF

    def c_row(r, cr2):
      (tv0, ti0), (tv1, ti1) = cr2
      row = r0 + r
      rowvec = jnp.full((L,), 0, jnp.int32) + row
      v0 = buf[r, pl.ds(coff, L)]
      v1 = buf[r, pl.ds(coff + L, L)]
      tv0, ti0 = _insert(list(tv0), list(ti0), v0, rowvec, True)
      tv1, ti1 = _insert(list(tv1), list(ti1), v1, rowvec, True)
      return ((tuple(tv0), tuple(ti0)), (tuple(tv1), tuple(ti1)))

    return lax.fori_loop(0, C_BUF, c_row, cr)

  c_copy(0, cbuf, sem0)
  c_copy(1, cbuf2, sem1)

  def c_pair(p, cr):
    ch0 = 2 * p
    c_drain(cbuf, sem0)
    cr = c_process(cbuf, ch0, cr)

    @pl.when(ch0 + 2 < nchC)
    def _():
      c_copy(ch0 + 2, cbuf, sem0)

    c_drain(cbuf2, sem1)
    cr = c_process(cbuf2, ch0 + 1, cr)

    @pl.when(ch0 + 3 < nchC)
    def _():
      c_copy(ch0 + 3, cbuf2, sem1)

    return cr

  carries = lax.fori_loop(
      0, nchC // 2, c_pair,
      tuple(_fresh_carry(C_K, True) for _ in range(2)))

  for g in range(2):
    tv, ti = carries[g]
    for j in range(C_K):
      pbufv[g, j, :] = tv[j]
      pbufi[g, j, :] = ti[j]
  pltpu.sync_copy(pbufv, shv.at[s])
  pltpu.sync_copy(pbufi, shi.at[s])
  plsc.subcore_barrier()

  @pl.when(s < 2)
  def _():
    g = s
    pltpu.sync_copy(shv, mrgv)
    pltpu.sync_copy(shi, mrgi)

    def m_tile(t, cr):
      tv = list(cr[0])
      ti = list(cr[1])
      for j in range(C_K):
        v = mrgv[t, g, j, :]
        vi = mrgi[t, g, j, :]
        tv, ti = _insert(tv, ti, v, vi, True)
      return (tuple(tv), tuple(ti))

    tv, ti = lax.fori_loop(0, NS, m_tile, _fresh_carry(C_K, True))
    for j in range(C_K):
      stgv[j, :] = tv[j]
      stgi[j, :] = ti[j]
    gout = c * 2 + g
    pltpu.sync_copy(stgv, outCv.at[gout])
    pltpu.sync_copy(stgi, outCi.at[gout])


@jax.jit
def kernel(v_0, v_1, v_2):
  v1r = v_1.reshape(B_ROWS, B_N)
  mesh = plsc.VectorSubcoreMesh(core_axis_name="c", subcore_axis_name="s")
  f32, i32 = jnp.float32, jnp.int32
  outs = pl.kernel(
      _sc_body,
      out_type=[
          jax.ShapeDtypeStruct((A_ROWS, L), f32),
          jax.ShapeDtypeStruct((A_ROWS, L), i32),
          jax.ShapeDtypeStruct((B_ROWS, L), f32),
          jax.ShapeDtypeStruct((B_ROWS, L), i32),
          jax.ShapeDtypeStruct((4, C_K, L), f32),
          jax.ShapeDtypeStruct((4, C_K, L), i32),
      ],
      mesh=mesh,
      compiler_params=pltpu.CompilerParams(
          needs_layout_passes=False, use_tc_tiling_on_sc=False),
      scratch_types=[
          pltpu.VMEM((B_BUF, B_N), f32),
          pltpu.VMEM((B_BUF, B_N), f32),
          pltpu.VMEM((C_BUF, C_COLS), f32),
          pltpu.VMEM((C_BUF, C_COLS), f32),
          pltpu.VMEM((B_RPT, L), f32),
          pltpu.VMEM((B_RPT, L), i32),
          pltpu.VMEM((A_RPT, L), f32),
          pltpu.VMEM((A_RPT, L), i32),
          pltpu.VMEM((2, C_K, L), f32),
          pltpu.VMEM((2, C_K, L), i32),
          pltpu.VMEM((NS, 2, C_K, L), f32),
          pltpu.VMEM((NS, 2, C_K, L), i32),
          pltpu.VMEM((C_K, L), f32),
          pltpu.VMEM((C_K, L), i32),
          pltpu.VMEM_SHARED((NS, 2, C_K, L), f32),
          pltpu.VMEM_SHARED((NS, 2, C_K, L), i32),
          pltpu.SemaphoreType.DMA,
          pltpu.SemaphoreType.DMA,
      ],
  )(v_0, v1r, v_2)
  outAv, outAi, outBv, outBi, outCv, outCi = outs
  v4 = outAv[:, :2]
  v5 = outAi[:, :2]
  v7 = outBv[:, :4].reshape(8, 16, 32, 4)
  v8 = outBi[:, :4].reshape(8, 16, 32, 4)
  v10 = outCv.transpose(1, 0, 2).reshape(C_K, C_COLS)
  v11 = outCi.transpose(1, 0, 2).reshape(C_K, C_COLS)
  return (v4, v5, v7, v8, v10, v11)
